# baseline (speedup 1.0000x reference)
import functools

import jax
import jax.numpy as jnp
from jax import lax
from jax.experimental import pallas as pl
from jax.experimental.pallas import tpu as pltpu
from jax.experimental.pallas import tpu_sc as plsc

_NC = 2
_NS = 16
_NW = _NC * _NS
_LANES = 16
_TV = 1024


def _prep_body(v_ref, u_ref, g_ref):
    voc = v_ref.shape[0]
    g_ref[0:voc, 0:voc] = lax.dot_general(
        v_ref[...], u_ref[...], (((1,), (1,)), ((), ())),
        preferred_element_type=jnp.float32,
        precision=lax.Precision.HIGHEST)


@functools.lru_cache(maxsize=None)
def _make_sc_gather(B, K1):
    bcols = B // _NW
    bpw = bcols * K1
    mesh = plsc.VectorSubcoreMesh(core_axis_name="c", subcore_axis_name="s")

    @functools.partial(
        pl.kernel, mesh=mesh,
        out_type=jax.ShapeDtypeStruct((_NW, 2 * _LANES), jnp.float32),
        scratch_types=[
            pltpu.VMEM((bcols,), jnp.int32),
            pltpu.VMEM((bpw,), jnp.int32),
            pltpu.VMEM((bpw,), jnp.int32),
            pltpu.VMEM((bpw,), jnp.float32),
            pltpu.VMEM((2 * _LANES,), jnp.float32),
            pltpu.SemaphoreType.DMA,
            pltpu.SemaphoreType.DMA,
        ])
    def sc_fn(g_hbm, cen_hbm, cnt_hbm, out_hbm,
              cen_v, cn_v, idx_v, val_v, st_v, isem, gsem):
        wid = lax.axis_index("s") * _NC + lax.axis_index("c")
        base = wid * bcols
        cps = [pltpu.async_copy(cen_hbm.at[pl.ds(base, bcols)], cen_v, isem)]
        cps += [
            pltpu.async_copy(cnt_hbm.at[k, pl.ds(base, bcols)],
                             cn_v.at[pl.ds(k * bcols, bcols)], isem)
            for k in range(K1)
        ]
        for cp in cps:
            cp.wait()

        for k in range(K1):
            for i in range(bcols // _LANES):
                o = k * bcols + i * _LANES
                c16 = cen_v[pl.ds(i * _LANES, _LANES)]
                n16 = cn_v[pl.ds(o, _LANES)]
                idx_v[pl.ds(o, _LANES)] = c16 * _TV + n16

        pltpu.async_copy(g_hbm.at[idx_v], val_v, gsem).wait()

        acc = jnp.zeros((_LANES,), jnp.float32)
        acc0 = jnp.zeros((_LANES,), jnp.float32)
        for i in range(bpw // _LANES):
            x = val_v[pl.ds(i * _LANES, _LANES)]
            acc = acc + jnp.exp(x)
            if i < bcols // _LANES:
                acc0 = acc0 + x
        st_v[pl.ds(0, _LANES)] = acc
        st_v[pl.ds(_LANES, _LANES)] = acc0
        pltpu.sync_copy(st_v, out_hbm.at[wid])

    return sc_fn


@functools.lru_cache(maxsize=None)
def _make_finalize(B):
    def _fin_body(p_ref, out_ref):
        s = jnp.sum(p_ref[:, 0:_LANES])
        l0 = jnp.sum(p_ref[:, _LANES:2 * _LANES])
        out_ref[...] = jnp.reshape(float(B) * jnp.log(s) - l0, (1, 1))

    return pl.pallas_call(
        _fin_body,
        out_shape=jax.ShapeDtypeStruct((1, 1), jnp.float32))


def kernel(V, U, centers, contexts_negs):
    voc, d = V.shape
    B = centers.shape[0]
    K1 = contexts_negs.shape[1]
    G = pl.pallas_call(
        _prep_body,
        out_shape=jax.ShapeDtypeStruct((_TV, _TV), jnp.float32),
    )(V, U)
    esum_lsum = _make_sc_gather(B, K1)(
        G.reshape(_TV * _TV), centers, contexts_negs.T)
    loss = _make_finalize(B)(esum_lsum)
    return loss[0, 0]
